# Initial kernel scaffold; baseline (speedup 1.0000x reference)
import jax
import jax.numpy as jnp
from jax.experimental import pallas as pl


def kernel(x, edge_index, W_gat, a_src, a_dst, b_gat, bn1_g, bn1_b, gin_eps, W_gin, b_gin, bn2_g, bn2_b):
    raise NotImplementedError("write your pallas kernel here")


# speedup vs baseline: 57.4697x; 57.4697x over previous
import functools

import jax
import jax.numpy as jnp
from jax import lax
from jax.experimental import pallas as pl
from jax.experimental.pallas import tpu as pltpu
from jax.experimental.pallas import tpu_sc as plsc

N = 10000
E = 320000
D = 128
H = 8
C = 16
HC = H * C

NCORES = 2
NSUB = 16
NTILES = NCORES * NSUB
EDGES_PER_TILE = E // NTILES
K = 80
NCHUNK = EDGES_PER_TILE // K
ROWS_MAIN = 624
ROWS_REM = N - NSUB * ROWS_MAIN


def _copy_tile_rows(s, src_ref, dst_ref):
    base = s * ROWS_MAIN
    pltpu.sync_copy(src_ref.at[pl.ds(base, ROWS_MAIN)],
                    dst_ref.at[pl.ds(base, ROWS_MAIN)])

    @pl.when(s == NSUB - 1)
    def _():
        pltpu.sync_copy(src_ref.at[pl.ds(NSUB * ROWS_MAIN, ROWS_REM)],
                        dst_ref.at[pl.ds(NSUB * ROWS_MAIN, ROWS_REM)])


def _lane_bcast(v, h):
    idx = jnp.full((16, 1), h, dtype=jnp.int32)
    dn = lax.GatherDimensionNumbers(
        offset_dims=(), collapsed_slice_dims=(0,), start_index_map=(0,))
    return lax.gather(v, idx, dn, (1,),
                      mode=lax.GatherScatterMode.PROMISE_IN_BOUNDS)


def _tc_pre_body(x_ref, w_ref, p1_ref, p2_ref, xw_ref, t1_ref, t2_ref):
    xw = jnp.dot(x_ref[...], w_ref[...], preferred_element_type=jnp.float32)
    xw_ref[...] = xw
    t1_ref[...] = jnp.dot(xw, p1_ref[...], preferred_element_type=jnp.float32)
    t2_ref[...] = jnp.dot(xw, p2_ref[...], preferred_element_type=jnp.float32)


def _tc_mid_body(num0_ref, num1_ref, den0_ref, den1_ref, xw_ref, t1_ref,
                 r_ref, rd_ref, bg_ref, g1_ref, b1_ref, h_ref):
    t1 = t1_ref[...]
    xw = xw_ref[...]
    e_self = t1 @ r_ref[...]
    e_self = jnp.where(e_self >= 0, e_self, 0.2 * e_self)
    e_self = jnp.exp(e_self)
    den = (den0_ref[...] + den1_ref[...]) @ rd_ref[...] + e_self
    num = num0_ref[...] + num1_ref[...] + e_self * xw
    gat = num / (den + 1e-16) + bg_ref[...]
    mu = jnp.mean(gat, axis=0, keepdims=True)
    var = jnp.mean(gat * gat, axis=0, keepdims=True) - mu * mu
    hv = (gat - mu) / jnp.sqrt(var + 1e-5) * g1_ref[...] + b1_ref[...]
    h_ref[...] = jnp.where(hv > 0, hv, jnp.exp(jnp.minimum(hv, 0.0)) - 1.0)


def _tc_post_body(h_ref, agg0_ref, agg1_ref, eps_ref, w_ref, bg_ref,
                  g2_ref, b2_ref, out_ref):
    g = (1.0 + eps_ref[0, 0]) * h_ref[...] + agg0_ref[...] + agg1_ref[...]
    z = jnp.dot(g, w_ref[...], preferred_element_type=jnp.float32) + bg_ref[...]
    mu = jnp.mean(z, axis=0, keepdims=True)
    var = jnp.mean(z * z, axis=0, keepdims=True) - mu * mu
    zz = (z - mu) / jnp.sqrt(var + 1e-5) * g2_ref[...] + b2_ref[...]
    out_ref[...] = jnp.where(zz > 0, zz, jnp.exp(jnp.minimum(zz, 0.0)) - 1.0)


def _make_sc_gat():
    mesh = plsc.VectorSubcoreMesh(core_axis_name="c", subcore_axis_name="s")

    @functools.partial(
        pl.kernel, mesh=mesh,
        compiler_params=pltpu.CompilerParams(use_tc_tiling_on_sc=False),
        out_type=[jax.ShapeDtypeStruct((NCORES, N, HC), jnp.float32),
                  jax.ShapeDtypeStruct((NCORES, N, 16), jnp.float32)],
        scratch_types=[
            pltpu.VMEM((NCHUNK, K), jnp.int32),
            pltpu.VMEM((NCHUNK, K), jnp.int32),
            pltpu.VMEM((K, 16), jnp.float32),
            pltpu.VMEM((K, 16), jnp.float32),
            pltpu.VMEM((K, HC), jnp.float32),
            pltpu.VMEM((K, 16), jnp.float32),
            pltpu.VMEM_SHARED((N, HC), jnp.float32),
            pltpu.VMEM_SHARED((N, 16), jnp.float32),
            pltpu.SemaphoreType.DMA,
            pltpu.SemaphoreType.DMA,
            pltpu.SemaphoreType.DMA,
        ])
    def sc_gat(xw_hbm, t1_hbm, t2_hbm, src_hbm, dst_hbm, z128_hbm, z16_hbm,
               num_out, den_out, src_v, dst_v, tsrc, tdst, xbuf, ebuf,
               num_sh, den_sh, sem0, sem1, sem2):
        c = lax.axis_index("c")
        s = lax.axis_index("s")
        tid = c * NSUB + s
        _copy_tile_rows(s, z128_hbm, num_sh)
        _copy_tile_rows(s, z16_hbm, den_sh)
        pltpu.sync_copy(src_hbm.at[tid], src_v)
        pltpu.sync_copy(dst_hbm.at[tid], dst_v)
        plsc.subcore_barrier()

        def chunk(k, _):
            sidx = src_v.at[k]
            didx = dst_v.at[k]
            cp0 = pltpu.async_copy(t1_hbm.at[sidx], tsrc, sem0)
            cp1 = pltpu.async_copy(t2_hbm.at[didx], tdst, sem1)
            cp2 = pltpu.async_copy(xw_hbm.at[sidx], xbuf, sem2)
            cp0.wait()
            cp1.wait()
            cp2.wait()

            def edge(i, _):
                vs = tsrc[i, :]
                vd = tdst[i, :]
                ep = vs + vd
                ep = jnp.where(ep >= 0, ep, 0.2 * ep)
                ev = jnp.exp(ep)
                ebuf[i, :] = ev
                for h in range(H):
                    sc = _lane_bcast(ev, h)
                    xbuf[i, pl.ds(h * 16, 16)] = xbuf[i, pl.ds(h * 16, 16)] * sc
                return 0

            lax.fori_loop(0, K, edge, 0)
            pltpu.sync_copy(ebuf, den_sh.at[didx], add=True)
            pltpu.sync_copy(xbuf, num_sh.at[didx], add=True)
            return 0

        lax.fori_loop(0, NCHUNK, chunk, 0)
        plsc.subcore_barrier()
        _copy_tile_rows(s, num_sh, num_out.at[c])
        _copy_tile_rows(s, den_sh, den_out.at[c])

    return sc_gat


def _make_sc_gin():
    mesh = plsc.VectorSubcoreMesh(core_axis_name="c", subcore_axis_name="s")

    @functools.partial(
        pl.kernel, mesh=mesh,
        out_type=[jax.ShapeDtypeStruct((NCORES, N, HC), jnp.float32)],
        scratch_types=[
            pltpu.VMEM((NCHUNK, K), jnp.int32),
            pltpu.VMEM((NCHUNK, K), jnp.int32),
            pltpu.VMEM((K, HC), jnp.float32),
            pltpu.VMEM_SHARED((N, HC), jnp.float32),
            pltpu.SemaphoreType.DMA,
        ])
    def sc_gin(h_hbm, src_hbm, dst_hbm, z128_hbm, agg_out,
               src_v, dst_v, xbuf, agg_sh, sem0):
        c = lax.axis_index("c")
        s = lax.axis_index("s")
        tid = c * NSUB + s
        _copy_tile_rows(s, z128_hbm, agg_sh)
        pltpu.sync_copy(src_hbm.at[tid], src_v)
        pltpu.sync_copy(dst_hbm.at[tid], dst_v)
        plsc.subcore_barrier()

        def chunk(k, _):
            pltpu.async_copy(h_hbm.at[src_v.at[k]], xbuf, sem0).wait()
            pltpu.sync_copy(xbuf, agg_sh.at[dst_v.at[k]], add=True)
            return 0

        lax.fori_loop(0, NCHUNK, chunk, 0)
        plsc.subcore_barrier()
        _copy_tile_rows(s, agg_sh, agg_out.at[c])

    return sc_gin


_sc_gat = _make_sc_gat()
_sc_gin = _make_sc_gin()


def _sds(shape):
    return jax.ShapeDtypeStruct(shape, jnp.float32)


_tc_pre = pl.pallas_call(
    _tc_pre_body, out_shape=[_sds((N, HC)), _sds((N, 16)), _sds((N, 16))])
_tc_mid = pl.pallas_call(_tc_mid_body, out_shape=_sds((N, HC)))
_tc_post = pl.pallas_call(_tc_post_body, out_shape=_sds((N, HC)))


def kernel(x, edge_index, W_gat, a_src, a_dst, b_gat, bn1_g, bn1_b,
           gin_eps, W_gin, b_gin, bn2_g, bn2_b):
    src3 = edge_index[0].reshape(NTILES, NCHUNK, K)
    dst3 = edge_index[1].reshape(NTILES, NCHUNK, K)

    eye_r = jnp.repeat(jnp.eye(H, dtype=jnp.float32), C, axis=0)
    asf = a_src.reshape(HC, 1)
    adf = a_dst.reshape(HC, 1)
    p1 = jnp.concatenate([eye_r * asf, eye_r * adf], axis=1)
    p2 = jnp.concatenate([eye_r * adf, eye_r * asf], axis=1)
    r_top = jnp.repeat(jnp.eye(H, dtype=jnp.float32), C, axis=1)
    r_mat = jnp.concatenate([r_top, r_top], axis=0)
    rd_mat = jnp.concatenate([r_top, jnp.zeros((H, HC), jnp.float32)], axis=0)
    z128 = jnp.zeros((N, HC), jnp.float32)
    z16 = jnp.zeros((N, 16), jnp.float32)

    xw, t1, t2 = _tc_pre(x, W_gat, p1, p2)
    num_p, den_p = _sc_gat(xw, t1, t2, src3, dst3, z128, z16)
    h1 = _tc_mid(num_p[0], num_p[1], den_p[0], den_p[1], xw, t1,
                 r_mat, rd_mat, b_gat.reshape(1, HC),
                 bn1_g.reshape(1, HC), bn1_b.reshape(1, HC))
    (agg_p,) = (_sc_gin(h1, src3, dst3, z128),)
    agg_p = agg_p[0] if isinstance(agg_p, (list, tuple)) else agg_p
    out = _tc_post(h1, agg_p[0], agg_p[1], gin_eps.reshape(1, 1), W_gin,
                   b_gin.reshape(1, HC), bn2_g.reshape(1, HC),
                   bn2_b.reshape(1, HC))
    return out


# speedup vs baseline: 77.8323x; 1.3543x over previous
import functools

import jax
import jax.numpy as jnp
from jax import lax
from jax.experimental import pallas as pl
from jax.experimental.pallas import tpu as pltpu
from jax.experimental.pallas import tpu_sc as plsc

N = 10000
E = 320000
D = 128
H = 8
C = 16
HC = H * C

NCORES = 2
NSUB = 16
NTILES = NCORES * NSUB
EDGES_PER_TILE = E // NTILES
K = 80
NCHUNK = EDGES_PER_TILE // K
ROWS_MAIN = 624
ROWS_REM = N - NSUB * ROWS_MAIN


def _copy_tile_rows(s, src_ref, dst_ref):
    base = s * ROWS_MAIN
    pltpu.sync_copy(src_ref.at[pl.ds(base, ROWS_MAIN)],
                    dst_ref.at[pl.ds(base, ROWS_MAIN)])

    @pl.when(s == NSUB - 1)
    def _():
        pltpu.sync_copy(src_ref.at[pl.ds(NSUB * ROWS_MAIN, ROWS_REM)],
                        dst_ref.at[pl.ds(NSUB * ROWS_MAIN, ROWS_REM)])


def _lane_bcast(v, h):
    idx = jnp.full((16, 1), h, dtype=jnp.int32)
    dn = lax.GatherDimensionNumbers(
        offset_dims=(), collapsed_slice_dims=(0,), start_index_map=(0,))
    return lax.gather(v, idx, dn, (1,),
                      mode=lax.GatherScatterMode.PROMISE_IN_BOUNDS)


def _tc_pre_body(x_ref, w_ref, p1_ref, p2_ref, xw_ref, t1_ref, t2_ref):
    xw = jnp.dot(x_ref[...], w_ref[...], preferred_element_type=jnp.float32)
    xw_ref[...] = xw
    t1_ref[...] = jnp.dot(xw, p1_ref[...], preferred_element_type=jnp.float32)
    t2_ref[...] = jnp.dot(xw, p2_ref[...], preferred_element_type=jnp.float32)


def _tc_mid_body(num0_ref, num1_ref, den0_ref, den1_ref, xw_ref, t1_ref,
                 r_ref, rd_ref, bg_ref, g1_ref, b1_ref, h_ref):
    t1 = t1_ref[...]
    xw = xw_ref[...]
    e_self = t1 @ r_ref[...]
    e_self = jnp.where(e_self >= 0, e_self, 0.2 * e_self)
    e_self = jnp.exp(e_self)
    den = (den0_ref[...] + den1_ref[...]) @ rd_ref[...] + e_self
    num = num0_ref[...] + num1_ref[...] + e_self * xw
    gat = num / (den + 1e-16) + bg_ref[...]
    mu = jnp.mean(gat, axis=0, keepdims=True)
    var = jnp.mean(gat * gat, axis=0, keepdims=True) - mu * mu
    hv = (gat - mu) / jnp.sqrt(var + 1e-5) * g1_ref[...] + b1_ref[...]
    h_ref[...] = jnp.where(hv > 0, hv, jnp.exp(jnp.minimum(hv, 0.0)) - 1.0)


def _tc_post_body(h_ref, agg0_ref, agg1_ref, eps_ref, w_ref, bg_ref,
                  g2_ref, b2_ref, out_ref):
    g = (1.0 + eps_ref[0, 0]) * h_ref[...] + agg0_ref[...] + agg1_ref[...]
    z = jnp.dot(g, w_ref[...], preferred_element_type=jnp.float32) + bg_ref[...]
    mu = jnp.mean(z, axis=0, keepdims=True)
    var = jnp.mean(z * z, axis=0, keepdims=True) - mu * mu
    zz = (z - mu) / jnp.sqrt(var + 1e-5) * g2_ref[...] + b2_ref[...]
    out_ref[...] = jnp.where(zz > 0, zz, jnp.exp(jnp.minimum(zz, 0.0)) - 1.0)


def _pipeline(nb, nchunk, wait_isrc, issue_isrc, wait_gather, issue_gather,
              wait_scatter, issue_scatter, compute):
    def step(m, p, do_a, do_c, do_d, do_e):
        pn = (p + 1) % nb
        if do_a:
            wait_isrc(pn)
        wait_gather(p)
        if do_c:
            wait_scatter(pn)
        if do_d:
            issue_gather(pn, m + 1)
        if do_e:
            issue_isrc(p, m + nb)
        compute(p)
        issue_scatter(p, m)

    issue_isrc(0, 0)
    wait_isrc(0)
    issue_gather(0, 0)
    for kk in range(1, nb):
        issue_isrc(kk % nb, kk)
    for m in range(nb):
        step(m, m % nb, True, m >= nb - 1, True, m + nb < nchunk)
    nturns = (nchunk - 2 * nb) // nb

    def turn(g, _):
        for j in range(nb):
            step(g * nb + j, j, True, True, True, True)
        return 0

    lax.fori_loop(1, nturns + 1, turn, 0)
    for m in range(nb + nturns * nb, nchunk):
        step(m, m % nb, m + 1 < nchunk, True, m + 1 < nchunk, m + nb < nchunk)
    for kpend in range(nchunk - nb + 1, nchunk):
        wait_scatter(kpend % nb)


def _make_sc_gat():
    mesh = plsc.VectorSubcoreMesh(core_axis_name="c", subcore_axis_name="s")
    nb = 2

    @functools.partial(
        pl.kernel, mesh=mesh,
        compiler_params=pltpu.CompilerParams(use_tc_tiling_on_sc=False),
        out_type=[jax.ShapeDtypeStruct((NCORES, N, HC), jnp.float32),
                  jax.ShapeDtypeStruct((NCORES, N, 16), jnp.float32)],
        scratch_types=(
            [pltpu.VMEM((NCHUNK, K), jnp.int32)] +
            [pltpu.VMEM((K,), jnp.int32)] * nb +
            [pltpu.VMEM((K, 16), jnp.float32)] * nb +
            [pltpu.VMEM((K, 16), jnp.float32)] * nb +
            [pltpu.VMEM((K, HC), jnp.float32)] * nb +
            [pltpu.VMEM((K, 16), jnp.float32)] * nb +
            [pltpu.VMEM_SHARED((N, HC), jnp.float32),
             pltpu.VMEM_SHARED((N, 16), jnp.float32)] +
            [pltpu.SemaphoreType.DMA] * (3 * nb)
        ))
    def sc_gat(xw_hbm, t1_hbm, t2_hbm, src_hbm, dst_hbm, z128_hbm, z16_hbm,
               num_out, den_out, dst_v, srcb0, srcb1,
               tsrc0, tsrc1, tdst0, tdst1, xbuf0, xbuf1, ebuf0, ebuf1,
               num_sh, den_sh, isem0, isem1, gsem0, gsem1, ssem0, ssem1):
        srcbs = (srcb0, srcb1)
        tsrcs = (tsrc0, tsrc1)
        tdsts = (tdst0, tdst1)
        xbufs = (xbuf0, xbuf1)
        ebufs = (ebuf0, ebuf1)
        isems = (isem0, isem1)
        gsems = (gsem0, gsem1)
        ssems = (ssem0, ssem1)
        c = lax.axis_index("c")
        s = lax.axis_index("s")
        tid = c * NSUB + s
        _copy_tile_rows(s, z128_hbm, num_sh)
        _copy_tile_rows(s, z16_hbm, den_sh)
        pltpu.sync_copy(dst_hbm.at[tid], dst_v)
        plsc.subcore_barrier()

        def issue_isrc(b, k):
            pltpu.async_copy(src_hbm.at[tid, k], srcbs[b], isems[b])

        def wait_isrc(b):
            pltpu.make_async_copy(src_hbm.at[0, 0], srcbs[b], isems[b]).wait()

        def issue_gather(b, k):
            sidx = srcbs[b]
            pltpu.async_copy(t1_hbm.at[sidx], tsrcs[b], gsems[b])
            pltpu.async_copy(t2_hbm.at[dst_v.at[k]], tdsts[b], gsems[b])
            pltpu.async_copy(xw_hbm.at[sidx], xbufs[b], gsems[b])

        def wait_gather(b):
            pltpu.make_async_copy(z16_hbm.at[pl.ds(0, K)], tsrcs[b], gsems[b]).wait()
            pltpu.make_async_copy(z16_hbm.at[pl.ds(0, K)], tdsts[b], gsems[b]).wait()
            pltpu.make_async_copy(z128_hbm.at[pl.ds(0, K)], xbufs[b], gsems[b]).wait()

        def issue_scatter(b, k):
            didx = dst_v.at[k]
            pltpu.async_copy(ebufs[b], den_sh.at[didx], ssems[b], add=True)
            pltpu.async_copy(xbufs[b], num_sh.at[didx], ssems[b], add=True)

        def wait_scatter(b):
            pltpu.make_async_copy(z16_hbm.at[pl.ds(0, K)], ebufs[b], ssems[b]).wait()
            pltpu.make_async_copy(z128_hbm.at[pl.ds(0, K)], xbufs[b], ssems[b]).wait()

        def compute(b):
            tsrc, tdst, xbuf, ebuf = tsrcs[b], tdsts[b], xbufs[b], ebufs[b]

            def edge(i, _):
                vs = tsrc[i, :]
                vd = tdst[i, :]
                ep = vs + vd
                ep = jnp.where(ep >= 0, ep, 0.2 * ep)
                ev = jnp.exp(ep)
                ebuf[i, :] = ev
                for h in range(H):
                    sc = _lane_bcast(ev, h)
                    xbuf[i, pl.ds(h * 16, 16)] = xbuf[i, pl.ds(h * 16, 16)] * sc
                return 0

            lax.fori_loop(0, K, edge, 0)

        _pipeline(nb, NCHUNK, wait_isrc, issue_isrc, wait_gather, issue_gather,
                  wait_scatter, issue_scatter, compute)

        plsc.subcore_barrier()
        _copy_tile_rows(s, num_sh, num_out.at[c])
        _copy_tile_rows(s, den_sh, den_out.at[c])

    return sc_gat


def _make_sc_gin():
    mesh = plsc.VectorSubcoreMesh(core_axis_name="c", subcore_axis_name="s")
    nb = 3

    @functools.partial(
        pl.kernel, mesh=mesh,
        compiler_params=pltpu.CompilerParams(use_tc_tiling_on_sc=False),
        out_type=[jax.ShapeDtypeStruct((NCORES, N, HC), jnp.float32)],
        scratch_types=(
            [pltpu.VMEM((NCHUNK, K), jnp.int32)] +
            [pltpu.VMEM((K,), jnp.int32)] * nb +
            [pltpu.VMEM((K, HC), jnp.float32)] * nb +
            [pltpu.VMEM_SHARED((N, HC), jnp.float32)] +
            [pltpu.SemaphoreType.DMA] * (3 * nb)
        ))
    def sc_gin(h_hbm, src_hbm, dst_hbm, z128_hbm, agg_out,
               dst_v, srcb0, srcb1, srcb2, xbuf0, xbuf1, xbuf2, agg_sh,
               isem0, isem1, isem2, gsem0, gsem1, gsem2, ssem0, ssem1, ssem2):
        srcbs = (srcb0, srcb1, srcb2)
        xbufs = (xbuf0, xbuf1, xbuf2)
        isems = (isem0, isem1, isem2)
        gsems = (gsem0, gsem1, gsem2)
        ssems = (ssem0, ssem1, ssem2)
        c = lax.axis_index("c")
        s = lax.axis_index("s")
        tid = c * NSUB + s
        _copy_tile_rows(s, z128_hbm, agg_sh)
        pltpu.sync_copy(dst_hbm.at[tid], dst_v)
        plsc.subcore_barrier()

        def issue_isrc(b, k):
            pltpu.async_copy(src_hbm.at[tid, k], srcbs[b], isems[b])

        def wait_isrc(b):
            pltpu.make_async_copy(src_hbm.at[0, 0], srcbs[b], isems[b]).wait()

        def issue_gather(b, k):
            pltpu.async_copy(h_hbm.at[srcbs[b]], xbufs[b], gsems[b])

        def wait_gather(b):
            pltpu.make_async_copy(z128_hbm.at[pl.ds(0, K)], xbufs[b], gsems[b]).wait()

        def issue_scatter(b, k):
            pltpu.async_copy(xbufs[b], agg_sh.at[dst_v.at[k]], ssems[b], add=True)

        def wait_scatter(b):
            pltpu.make_async_copy(z128_hbm.at[pl.ds(0, K)], xbufs[b], ssems[b]).wait()

        _pipeline(nb, NCHUNK, wait_isrc, issue_isrc, wait_gather, issue_gather,
                  wait_scatter, issue_scatter, lambda b: None)

        plsc.subcore_barrier()
        _copy_tile_rows(s, agg_sh, agg_out.at[c])

    return sc_gin


_sc_gat = _make_sc_gat()
_sc_gin = _make_sc_gin()


def _sds(shape):
    return jax.ShapeDtypeStruct(shape, jnp.float32)


_tc_pre = pl.pallas_call(
    _tc_pre_body, out_shape=[_sds((N, HC)), _sds((N, 16)), _sds((N, 16))])
_tc_mid = pl.pallas_call(_tc_mid_body, out_shape=_sds((N, HC)))
_tc_post = pl.pallas_call(_tc_post_body, out_shape=_sds((N, HC)))


def kernel(x, edge_index, W_gat, a_src, a_dst, b_gat, bn1_g, bn1_b,
           gin_eps, W_gin, b_gin, bn2_g, bn2_b):
    src3 = edge_index[0].reshape(NTILES, NCHUNK, K)
    dst3 = edge_index[1].reshape(NTILES, NCHUNK, K)

    eye_r = jnp.repeat(jnp.eye(H, dtype=jnp.float32), C, axis=0)
    asf = a_src.reshape(HC, 1)
    adf = a_dst.reshape(HC, 1)
    p1 = jnp.concatenate([eye_r * asf, eye_r * adf], axis=1)
    p2 = jnp.concatenate([eye_r * adf, eye_r * asf], axis=1)
    r_top = jnp.repeat(jnp.eye(H, dtype=jnp.float32), C, axis=1)
    r_mat = jnp.concatenate([r_top, r_top], axis=0)
    rd_mat = jnp.concatenate([r_top, jnp.zeros((H, HC), jnp.float32)], axis=0)
    z128 = jnp.zeros((N, HC), jnp.float32)
    z16 = jnp.zeros((N, 16), jnp.float32)

    xw, t1, t2 = _tc_pre(x, W_gat, p1, p2)
    num_p, den_p = _sc_gat(xw, t1, t2, src3, dst3, z128, z16)
    h1 = _tc_mid(num_p[0], num_p[1], den_p[0], den_p[1], xw, t1,
                 r_mat, rd_mat, b_gat.reshape(1, HC),
                 bn1_g.reshape(1, HC), bn1_b.reshape(1, HC))
    (agg_p,) = (_sc_gin(h1, src3, dst3, z128),)
    agg_p = agg_p[0] if isinstance(agg_p, (list, tuple)) else agg_p
    out = _tc_post(h1, agg_p[0], agg_p[1], gin_eps.reshape(1, 1), W_gin,
                   b_gin.reshape(1, HC), bn2_g.reshape(1, HC),
                   bn2_b.reshape(1, HC))
    return out


# speedup vs baseline: 87.3430x; 1.1222x over previous
import functools

import jax
import jax.numpy as jnp
from jax import lax
from jax.experimental import pallas as pl
from jax.experimental.pallas import tpu as pltpu
from jax.experimental.pallas import tpu_sc as plsc

N = 10000
E = 320000
D = 128
H = 8
C = 16
HC = H * C

NCORES = 2
NSUB = 16
NTILES = NCORES * NSUB
EDGES_PER_TILE = E // NTILES
K = 80
NCHUNK = EDGES_PER_TILE // K
ROWS_MAIN = 624
ROWS_REM = N - NSUB * ROWS_MAIN


def _copy_tile_rows(s, src_ref, dst_ref):
    base = s * ROWS_MAIN
    pltpu.sync_copy(src_ref.at[pl.ds(base, ROWS_MAIN)],
                    dst_ref.at[pl.ds(base, ROWS_MAIN)])

    @pl.when(s == NSUB - 1)
    def _():
        pltpu.sync_copy(src_ref.at[pl.ds(NSUB * ROWS_MAIN, ROWS_REM)],
                        dst_ref.at[pl.ds(NSUB * ROWS_MAIN, ROWS_REM)])


def _lane_bcast(v, h):
    idx = jnp.full((16, 1), h, dtype=jnp.int32)
    dn = lax.GatherDimensionNumbers(
        offset_dims=(), collapsed_slice_dims=(0,), start_index_map=(0,))
    return lax.gather(v, idx, dn, (1,),
                      mode=lax.GatherScatterMode.PROMISE_IN_BOUNDS)


def _tc_pre_body(x_ref, w_ref, p1_ref, p2_ref, xw_ref, t1_ref, t2_ref):
    xw = jnp.dot(x_ref[...], w_ref[...], preferred_element_type=jnp.float32)
    xw_ref[...] = xw
    t1_ref[...] = jnp.dot(xw, p1_ref[...], preferred_element_type=jnp.float32)
    t2_ref[...] = jnp.dot(xw, p2_ref[...], preferred_element_type=jnp.float32)


def _tc_mid_body(num0_ref, num1_ref, den0_ref, den1_ref, xw_ref, t1_ref,
                 r_ref, rd_ref, bg_ref, g1_ref, b1_ref, h_ref):
    t1 = t1_ref[...]
    xw = xw_ref[...]
    e_self = t1 @ r_ref[...]
    e_self = jnp.where(e_self >= 0, e_self, 0.2 * e_self)
    e_self = jnp.exp(e_self)
    den = (den0_ref[...] + den1_ref[...]) @ rd_ref[...] + e_self
    num = num0_ref[...] + num1_ref[...] + e_self * xw
    gat = num / (den + 1e-16) + bg_ref[...]
    mu = jnp.mean(gat, axis=0, keepdims=True)
    var = jnp.mean(gat * gat, axis=0, keepdims=True) - mu * mu
    hv = (gat - mu) / jnp.sqrt(var + 1e-5) * g1_ref[...] + b1_ref[...]
    h_ref[...] = jnp.where(hv > 0, hv, jnp.exp(jnp.minimum(hv, 0.0)) - 1.0)


def _tc_post_body(h_ref, agg0_ref, agg1_ref, eps_ref, w_ref, bg_ref,
                  g2_ref, b2_ref, out_ref):
    g = (1.0 + eps_ref[0, 0]) * h_ref[...] + agg0_ref[...] + agg1_ref[...]
    z = jnp.dot(g, w_ref[...], preferred_element_type=jnp.float32) + bg_ref[...]
    mu = jnp.mean(z, axis=0, keepdims=True)
    var = jnp.mean(z * z, axis=0, keepdims=True) - mu * mu
    zz = (z - mu) / jnp.sqrt(var + 1e-5) * g2_ref[...] + b2_ref[...]
    out_ref[...] = jnp.where(zz > 0, zz, jnp.exp(jnp.minimum(zz, 0.0)) - 1.0)


def _pipeline(nb, nchunk, wait_isrc, issue_isrc, wait_gather, issue_gather,
              wait_scatter, issue_scatter, compute):
    def step(m, p, do_a, do_c, do_d, do_e):
        pn = (p + 1) % nb
        if do_a:
            wait_isrc(pn)
        wait_gather(p)
        compute(p)
        issue_scatter(p, m)
        if do_c:
            wait_scatter(pn)
        if do_d:
            issue_gather(pn, m + 1)
        if do_e:
            issue_isrc(p, m + nb)

    issue_isrc(0, 0)
    wait_isrc(0)
    issue_gather(0, 0)
    for kk in range(1, nb):
        issue_isrc(kk % nb, kk)
    for m in range(nb):
        step(m, m % nb, True, m >= nb - 1, True, m + nb < nchunk)
    nturns = (nchunk - 2 * nb) // nb

    def turn(g, _):
        for j in range(nb):
            step(g * nb + j, j, True, True, True, True)
        return 0

    lax.fori_loop(1, nturns + 1, turn, 0)
    for m in range(nb + nturns * nb, nchunk):
        step(m, m % nb, m + 1 < nchunk, True, m + 1 < nchunk, m + nb < nchunk)
    for kpend in range(nchunk - nb + 1, nchunk):
        wait_scatter(kpend % nb)


def _make_sc_gat():
    mesh = plsc.VectorSubcoreMesh(core_axis_name="c", subcore_axis_name="s")
    nb = 2

    @functools.partial(
        pl.kernel, mesh=mesh,
        compiler_params=pltpu.CompilerParams(use_tc_tiling_on_sc=False),
        out_type=[jax.ShapeDtypeStruct((NCORES, N, HC), jnp.float32),
                  jax.ShapeDtypeStruct((NCORES, N, 16), jnp.float32)],
        scratch_types=(
            [pltpu.VMEM((NCHUNK, K), jnp.int32)] +
            [pltpu.VMEM((K,), jnp.int32)] * nb +
            [pltpu.VMEM((K, 16), jnp.float32)] * nb +
            [pltpu.VMEM((K, 16), jnp.float32)] * nb +
            [pltpu.VMEM((K, HC), jnp.float32)] * nb +
            [pltpu.VMEM((K, 16), jnp.float32)] * nb +
            [pltpu.VMEM_SHARED((N, HC), jnp.float32),
             pltpu.VMEM_SHARED((N, 16), jnp.float32)] +
            [pltpu.SemaphoreType.DMA] * (3 * nb)
        ))
    def sc_gat(xw_hbm, t1_hbm, t2_hbm, src_hbm, dst_hbm, z128_hbm, z16_hbm,
               num_out, den_out, dst_v, srcb0, srcb1,
               tsrc0, tsrc1, tdst0, tdst1, xbuf0, xbuf1, ebuf0, ebuf1,
               num_sh, den_sh, isem0, isem1, gsem0, gsem1, ssem0, ssem1):
        srcbs = (srcb0, srcb1)
        tsrcs = (tsrc0, tsrc1)
        tdsts = (tdst0, tdst1)
        xbufs = (xbuf0, xbuf1)
        ebufs = (ebuf0, ebuf1)
        isems = (isem0, isem1)
        gsems = (gsem0, gsem1)
        ssems = (ssem0, ssem1)
        c = lax.axis_index("c")
        s = lax.axis_index("s")
        tid = c * NSUB + s
        _copy_tile_rows(s, z128_hbm, num_sh)
        _copy_tile_rows(s, z16_hbm, den_sh)
        pltpu.sync_copy(dst_hbm.at[tid], dst_v)
        plsc.subcore_barrier()

        def issue_isrc(b, k):
            pltpu.async_copy(src_hbm.at[tid, k], srcbs[b], isems[b])

        def wait_isrc(b):
            pltpu.make_async_copy(src_hbm.at[0, 0], srcbs[b], isems[b]).wait()

        def issue_gather(b, k):
            sidx = srcbs[b]
            pltpu.async_copy(t1_hbm.at[sidx], tsrcs[b], gsems[b])
            pltpu.async_copy(t2_hbm.at[dst_v.at[k]], tdsts[b], gsems[b])
            pltpu.async_copy(xw_hbm.at[sidx], xbufs[b], gsems[b])

        def wait_gather(b):
            pltpu.make_async_copy(z16_hbm.at[pl.ds(0, K)], tsrcs[b], gsems[b]).wait()
            pltpu.make_async_copy(z16_hbm.at[pl.ds(0, K)], tdsts[b], gsems[b]).wait()
            pltpu.make_async_copy(z128_hbm.at[pl.ds(0, K)], xbufs[b], gsems[b]).wait()

        def issue_scatter(b, k):
            didx = dst_v.at[k]
            pltpu.async_copy(ebufs[b], den_sh.at[didx], ssems[b], add=True)
            pltpu.async_copy(xbufs[b], num_sh.at[didx], ssems[b], add=True)

        def wait_scatter(b):
            pltpu.make_async_copy(z16_hbm.at[pl.ds(0, K)], ebufs[b], ssems[b]).wait()
            pltpu.make_async_copy(z128_hbm.at[pl.ds(0, K)], xbufs[b], ssems[b]).wait()

        def compute(b):
            tsrc, tdst, xbuf, ebuf = tsrcs[b], tdsts[b], xbufs[b], ebufs[b]

            @plsc.parallel_loop(0, K, unroll=4)
            def edge(i):
                vs = tsrc[i, :]
                vd = tdst[i, :]
                ep = vs + vd
                ep = jnp.where(ep >= 0, ep, 0.2 * ep)
                ev = jnp.exp(ep)
                ebuf[i, :] = ev
                for h in range(H):
                    sc = _lane_bcast(ev, h)
                    xbuf[i, pl.ds(h * 16, 16)] = xbuf[i, pl.ds(h * 16, 16)] * sc

        _pipeline(nb, NCHUNK, wait_isrc, issue_isrc, wait_gather, issue_gather,
                  wait_scatter, issue_scatter, compute)

        plsc.subcore_barrier()
        _copy_tile_rows(s, num_sh, num_out.at[c])
        _copy_tile_rows(s, den_sh, den_out.at[c])

    return sc_gat


def _make_sc_gin():
    mesh = plsc.VectorSubcoreMesh(core_axis_name="c", subcore_axis_name="s")
    nb = 3

    @functools.partial(
        pl.kernel, mesh=mesh,
        compiler_params=pltpu.CompilerParams(use_tc_tiling_on_sc=False),
        out_type=[jax.ShapeDtypeStruct((NCORES, N, HC), jnp.float32)],
        scratch_types=(
            [pltpu.VMEM((NCHUNK, K), jnp.int32)] +
            [pltpu.VMEM((K,), jnp.int32)] * nb +
            [pltpu.VMEM((K, HC), jnp.float32)] * nb +
            [pltpu.VMEM_SHARED((N, HC), jnp.float32)] +
            [pltpu.SemaphoreType.DMA] * (3 * nb)
        ))
    def sc_gin(h_hbm, src_hbm, dst_hbm, z128_hbm, agg_out,
               dst_v, srcb0, srcb1, srcb2, xbuf0, xbuf1, xbuf2, agg_sh,
               isem0, isem1, isem2, gsem0, gsem1, gsem2, ssem0, ssem1, ssem2):
        srcbs = (srcb0, srcb1, srcb2)
        xbufs = (xbuf0, xbuf1, xbuf2)
        isems = (isem0, isem1, isem2)
        gsems = (gsem0, gsem1, gsem2)
        ssems = (ssem0, ssem1, ssem2)
        c = lax.axis_index("c")
        s = lax.axis_index("s")
        tid = c * NSUB + s
        _copy_tile_rows(s, z128_hbm, agg_sh)
        pltpu.sync_copy(dst_hbm.at[tid], dst_v)
        plsc.subcore_barrier()

        def issue_isrc(b, k):
            pltpu.async_copy(src_hbm.at[tid, k], srcbs[b], isems[b])

        def wait_isrc(b):
            pltpu.make_async_copy(src_hbm.at[0, 0], srcbs[b], isems[b]).wait()

        def issue_gather(b, k):
            pltpu.async_copy(h_hbm.at[srcbs[b]], xbufs[b], gsems[b])

        def wait_gather(b):
            pltpu.make_async_copy(z128_hbm.at[pl.ds(0, K)], xbufs[b], gsems[b]).wait()

        def issue_scatter(b, k):
            pltpu.async_copy(xbufs[b], agg_sh.at[dst_v.at[k]], ssems[b], add=True)

        def wait_scatter(b):
            pltpu.make_async_copy(z128_hbm.at[pl.ds(0, K)], xbufs[b], ssems[b]).wait()

        _pipeline(nb, NCHUNK, wait_isrc, issue_isrc, wait_gather, issue_gather,
                  wait_scatter, issue_scatter, lambda b: None)

        plsc.subcore_barrier()
        _copy_tile_rows(s, agg_sh, agg_out.at[c])

    return sc_gin


_sc_gat = _make_sc_gat()
_sc_gin = _make_sc_gin()


def _sds(shape):
    return jax.ShapeDtypeStruct(shape, jnp.float32)


_tc_pre = pl.pallas_call(
    _tc_pre_body, out_shape=[_sds((N, HC)), _sds((N, 16)), _sds((N, 16))])
_tc_mid = pl.pallas_call(_tc_mid_body, out_shape=_sds((N, HC)))
_tc_post = pl.pallas_call(_tc_post_body, out_shape=_sds((N, HC)))


def kernel(x, edge_index, W_gat, a_src, a_dst, b_gat, bn1_g, bn1_b,
           gin_eps, W_gin, b_gin, bn2_g, bn2_b):
    src3 = edge_index[0].reshape(NTILES, NCHUNK, K)
    dst3 = edge_index[1].reshape(NTILES, NCHUNK, K)

    eye_r = jnp.repeat(jnp.eye(H, dtype=jnp.float32), C, axis=0)
    asf = a_src.reshape(HC, 1)
    adf = a_dst.reshape(HC, 1)
    p1 = jnp.concatenate([eye_r * asf, eye_r * adf], axis=1)
    p2 = jnp.concatenate([eye_r * adf, eye_r * asf], axis=1)
    r_top = jnp.repeat(jnp.eye(H, dtype=jnp.float32), C, axis=1)
    r_mat = jnp.concatenate([r_top, r_top], axis=0)
    rd_mat = jnp.concatenate([r_top, jnp.zeros((H, HC), jnp.float32)], axis=0)
    z128 = jnp.zeros((N, HC), jnp.float32)
    z16 = jnp.zeros((N, 16), jnp.float32)

    xw, t1, t2 = _tc_pre(x, W_gat, p1, p2)
    num_p, den_p = _sc_gat(xw, t1, t2, src3, dst3, z128, z16)
    h1 = _tc_mid(num_p[0], num_p[1], den_p[0], den_p[1], xw, t1,
                 r_mat, rd_mat, b_gat.reshape(1, HC),
                 bn1_g.reshape(1, HC), bn1_b.reshape(1, HC))
    (agg_p,) = (_sc_gin(h1, src3, dst3, z128),)
    agg_p = agg_p[0] if isinstance(agg_p, (list, tuple)) else agg_p
    out = _tc_post(h1, agg_p[0], agg_p[1], gin_eps.reshape(1, 1), W_gin,
                   b_gin.reshape(1, HC), bn2_g.reshape(1, HC),
                   bn2_b.reshape(1, HC))
    return out
